# baseline (speedup 1.0000x reference)
import functools

import jax
import jax.numpy as jnp
from jax import lax
from jax.experimental import pallas as pl
from jax.experimental.pallas import tpu as pltpu
from jax.experimental.pallas import tpu_sc as plsc

_NW = 32
_ROWS = 49152
_RPW = _ROWS // _NW
_C = 48
_NCH = _RPW // _C
_D = 1024
_R0 = 0
_R1 = 16385


def _sc_body(y_hbm, m_hbm, o_hbm, buf, fbuf, mbuf, gsem, ssem):
    wid = lax.axis_index("s") * 2 + lax.axis_index("c")
    base = wid * _RPW

    def g_copy(ci, slot):
        return pltpu.make_async_copy(
            y_hbm.at[pl.ds(base + ci * _C, _C), :], buf.at[slot], gsem
        )

    def s_copy(ci, slot):
        return pltpu.make_async_copy(
            buf.at[slot], o_hbm.at[pl.ds(base + ci * _C, _C), :], ssem
        )

    g_copy(0, 0).start()

    def chunk_pair(k, _):
        for b in range(2):
            ci = 2 * k + b
            slot, oslot = b, 1 - b

            @pl.when(ci >= 1)
            def _():
                s_copy(ci - 1, oslot).wait()

            @pl.when(ci + 1 < _NCH)
            def _():
                g_copy(ci + 1, oslot).start()

            g_copy(ci, slot).wait()
            s_copy(ci, slot).start()
        return _

    lax.fori_loop(0, _NCH // 2, chunk_pair, None)
    s_copy(_NCH - 1, 1).wait()

    for owner, band, moff in ((_R0 // _RPW, (_R0 // 8) * 8, 0), (_R1 // _RPW, (_R1 // 8) * 8, 8)):
        @pl.when(wid == owner)
        def _():
            pltpu.sync_copy(o_hbm.at[pl.ds(band, 8), :], fbuf)
            pltpu.sync_copy(m_hbm.at[pl.ds(moff, 8), :], mbuf)
            for r in range(8):
                def add16(j, _):
                    sl = pl.ds(j * 16, 16)
                    fbuf[r, sl] = fbuf[r, sl] + mbuf[r, sl]
                    return _
                lax.fori_loop(0, _D // 16, add16, None)
            pltpu.sync_copy(fbuf, o_hbm.at[pl.ds(band, 8), :])


@functools.partial(jax.jit, static_argnums=())
def _sc_call(y, m):
    mesh = plsc.VectorSubcoreMesh(core_axis_name="c", subcore_axis_name="s")
    return pl.kernel(
        _sc_body,
        out_type=jax.ShapeDtypeStruct((_ROWS, _D), jnp.float32),
        mesh=mesh,
        scratch_types=[
            pltpu.VMEM((2, _C, _D), jnp.float32),
            pltpu.VMEM((8, _D), jnp.float32),
            pltpu.VMEM((8, _D), jnp.float32),
            pltpu.SemaphoreType.DMA,
            pltpu.SemaphoreType.DMA,
        ],
    )(y, m)


def kernel(x):
    n, s, d = x.shape
    y = jnp.transpose(x, (1, 0, 2)).reshape(s * n, d)
    m = (
        jnp.zeros((16, d), jnp.float32)
        .at[_R0 % 8, :].set(1.0)
        .at[8 + _R1 % 8, :].set(1.0)
    )
    out = _sc_call(y, m)
    return jnp.transpose(out.reshape(s, n, d), (1, 0, 2))


# speedup vs baseline: 1.0037x; 1.0037x over previous
import functools

import jax
import jax.numpy as jnp
from jax import lax
from jax.experimental import pallas as pl
from jax.experimental.pallas import tpu as pltpu
from jax.experimental.pallas import tpu_sc as plsc

_NW = 32
_ROWS = 49152
_RPW = _ROWS // _NW
_C = 32
_NBUF = 3
_NCH = _RPW // _C
_D = 1024
_R0 = 0
_R1 = 16385


def _sc_body(y_hbm, m_hbm, o_hbm, buf, fbuf, mbuf, gsem, ssem):
    wid = lax.axis_index("s") * 2 + lax.axis_index("c")
    base = wid * _RPW

    def g_copy(ci, slot):
        return pltpu.make_async_copy(
            y_hbm.at[pl.ds(base + ci * _C, _C), :], buf.at[slot], gsem
        )

    def s_copy(ci, slot):
        return pltpu.make_async_copy(
            buf.at[slot], o_hbm.at[pl.ds(base + ci * _C, _C), :], ssem
        )

    g_copy(0, 0).start()

    def chunk_group(k, _):
        for b in range(_NBUF):
            ci = _NBUF * k + b
            slot = b

            @pl.when(ci >= _NBUF - 1)
            def _():
                s_copy(ci - (_NBUF - 1), (ci + 1) % _NBUF).wait()

            @pl.when(ci + 1 < _NCH)
            def _():
                g_copy(ci + 1, (ci + 1) % _NBUF).start()

            g_copy(ci, slot).wait()
            s_copy(ci, slot).start()
        return _

    lax.fori_loop(0, _NCH // _NBUF, chunk_group, None)
    for ci in range(_NCH - (_NBUF - 1), _NCH):
        s_copy(ci, ci % _NBUF).wait()

    for owner, band, moff in ((_R0 // _RPW, (_R0 // 8) * 8, 0), (_R1 // _RPW, (_R1 // 8) * 8, 8)):
        @pl.when(wid == owner)
        def _():
            pltpu.sync_copy(o_hbm.at[pl.ds(band, 8), :], fbuf)
            pltpu.sync_copy(m_hbm.at[pl.ds(moff, 8), :], mbuf)
            for r in range(8):
                def add16(j, _):
                    sl = pl.ds(j * 16, 16)
                    fbuf[r, sl] = fbuf[r, sl] + mbuf[r, sl]
                    return _
                lax.fori_loop(0, _D // 16, add16, None)
            pltpu.sync_copy(fbuf, o_hbm.at[pl.ds(band, 8), :])


@functools.partial(jax.jit, static_argnums=())
def _sc_call(y, m):
    mesh = plsc.VectorSubcoreMesh(core_axis_name="c", subcore_axis_name="s")
    return pl.kernel(
        _sc_body,
        out_type=jax.ShapeDtypeStruct((_ROWS, _D), jnp.float32),
        mesh=mesh,
        scratch_types=[
            pltpu.VMEM((_NBUF, _C, _D), jnp.float32),
            pltpu.VMEM((8, _D), jnp.float32),
            pltpu.VMEM((8, _D), jnp.float32),
            pltpu.SemaphoreType.DMA,
            pltpu.SemaphoreType.DMA,
        ],
    )(y, m)


def kernel(x):
    n, s, d = x.shape
    y = jnp.transpose(x, (1, 0, 2)).reshape(s * n, d)
    m = (
        jnp.zeros((16, d), jnp.float32)
        .at[_R0 % 8, :].set(1.0)
        .at[8 + _R1 % 8, :].set(1.0)
    )
    out = _sc_call(y, m)
    return jnp.transpose(out.reshape(s, n, d), (1, 0, 2))


# speedup vs baseline: 1.2894x; 1.2847x over previous
import jax
import jax.numpy as jnp
from jax.experimental import pallas as pl

_BLK = 2048
_R0 = 0
_R1 = 16385


def _copy_scatter_body(x_ref, o_ref):
    i = pl.program_id(0)
    o_ref[...] = x_ref[...]

    @pl.when(i == _R0 // _BLK)
    def _():
        r = _R0 % _BLK
        o_ref[pl.ds(r, 1), :] = o_ref[pl.ds(r, 1), :] + jnp.float32(1.0)

    @pl.when(i == _R1 // _BLK)
    def _():
        r = _R1 % _BLK
        o_ref[pl.ds(r, 1), :] = o_ref[pl.ds(r, 1), :] + jnp.float32(1.0)


def kernel(x):
    n, s, d = x.shape
    y = jnp.transpose(x, (1, 0, 2)).reshape(s * n, d)
    out = pl.pallas_call(
        _copy_scatter_body,
        out_shape=jax.ShapeDtypeStruct((s * n, d), x.dtype),
        grid=(s * n // _BLK,),
        in_specs=[pl.BlockSpec((_BLK, d), lambda i: (i, 0))],
        out_specs=pl.BlockSpec((_BLK, d), lambda i: (i, 0)),
    )(y)
    return jnp.transpose(out.reshape(s, n, d), (1, 0, 2))
